# Initial kernel scaffold; baseline (speedup 1.0000x reference)
import jax
import jax.numpy as jnp
from jax.experimental import pallas as pl


def kernel(x, edge_index, W_in, b_in, W1, b1, W2, b2, W_ih, W_hh, b_ih, b_hh, Wo1, bo1, Wo2, bo2):
    raise NotImplementedError("write your pallas kernel here")


# speedup vs baseline: 2.2492x; 2.2492x over previous
import jax
import jax.numpy as jnp
from jax.experimental import pallas as pl

_B = 32
_N = 81
_E = 1620
_H = 128
_M = 128
_NSTEPS = 8
_D = 64
_EPAD = _D * _N


def _rrn_body(xn_ref, gs_ref, gr_ref, mask_ref, degb2_ref,
              w_in_ref, b_in_ref, w1_ref, b1_ref, w2_ref,
              w_ih_ref, w_hh_ref, b_ih_ref, b_hh_ref,
              wo1_ref, bo1_ref, wo2_ref, bo2_ref, out_ref):
    f32 = jnp.float32
    bf16 = jnp.bfloat16
    gs = gs_ref[...]
    gr = gr_ref[...]
    mask = mask_ref[...]
    w1 = w1_ref[...]
    b1 = b1_ref[...]
    w2 = w2_ref[...]
    w_ih = w_ih_ref[...]
    w_hh = w_hh_ref[...]
    b_ih = b_ih_ref[...]
    b_hh = b_hh_ref[...]
    degb2 = degb2_ref[...]

    h = jnp.dot(xn_ref[...], w_in_ref[...], preferred_element_type=f32) + b_in_ref[...]

    for _ in range(_NSTEPS):
        hb = h.astype(bf16)
        hb3 = hb.reshape(_B, _N, _H)
        s_rows = []
        for b in range(_B):
            hbb = hb3[b]
            hs = jnp.dot(gs, hbb, preferred_element_type=f32)
            hr = jnp.dot(gr, hbb, preferred_element_type=f32)
            msg_in = jnp.concatenate([hs, hr], axis=1).astype(bf16)
            pre = jnp.dot(msg_in, w1, preferred_element_type=f32) + b1
            t = jnp.maximum(pre, 0.0).astype(bf16)
            msgs = jnp.dot(t, w2, preferred_element_type=f32) * mask
            m3 = msgs.reshape(_D, _N, _M)
            acc = m3[0]
            for d in range(1, _D):
                acc = acc + m3[d]
            s_rows.append(acc)
        s = jnp.concatenate(s_rows, axis=0)
        agg = s + degb2
        gi = jnp.dot(agg.astype(bf16), w_ih, preferred_element_type=f32) + b_ih
        gh = jnp.dot(hb, w_hh, preferred_element_type=f32) + b_hh
        r = jax.nn.sigmoid(gi[:, :_H] + gh[:, :_H])
        z = jax.nn.sigmoid(gi[:, _H:2 * _H] + gh[:, _H:2 * _H])
        n = jnp.tanh(gi[:, 2 * _H:] + r * gh[:, 2 * _H:])
        h = (1.0 - z) * n + z * h

    o1 = jnp.maximum(jnp.dot(h.astype(bf16), wo1_ref[...], preferred_element_type=f32) + bo1_ref[...], 0.0)
    out_ref[...] = jnp.dot(o1.astype(bf16), wo2_ref[...], preferred_element_type=f32) + bo2_ref[...]


def kernel(x, edge_index, W_in, b_in, W1, b1, W2, b2, W_ih, W_hh, b_ih, b_hh, Wo1, bo1, Wo2, bo2):
    f32 = jnp.float32
    bf16 = jnp.bfloat16
    senders = edge_index[0]
    receivers = edge_index[1]

    order = jnp.argsort(receivers, stable=True)
    sorted_recv = receivers[order]
    deg = jnp.bincount(receivers, length=_N)
    seg_start = jnp.cumsum(deg) - deg
    rank_sorted = jnp.arange(_E, dtype=jnp.int32) - seg_start[sorted_recv].astype(jnp.int32)
    rank = jnp.zeros((_E,), jnp.int32).at[order].set(rank_sorted)
    q = rank * _N + receivers

    oh_s = jax.nn.one_hot(senders, _N, dtype=f32)
    oh_r = jax.nn.one_hot(receivers, _N, dtype=f32)
    gs_pad = jnp.zeros((_EPAD, _N), f32).at[q].set(oh_s).astype(bf16)
    gr_pad = jnp.zeros((_EPAD, _N), f32).at[q].set(oh_r).astype(bf16)
    mask = jnp.zeros((_EPAD, 1), f32).at[q, 0].set(1.0)

    deg_rows = jnp.tile(deg.astype(f32), _B)[:, None]
    degb2 = deg_rows * b2[None, :]

    xn = x.reshape(_B * _N, x.shape[-1]).astype(bf16)
    wo2p = jnp.pad(Wo2, ((0, 0), (0, _H - Wo2.shape[1]))).astype(bf16)
    bo2p = jnp.pad(bo2, (0, _H - bo2.shape[0]))[None, :]

    out = pl.pallas_call(
        _rrn_body,
        out_shape=jax.ShapeDtypeStruct((_B * _N, _H), f32),
    )(xn, gs_pad, gr_pad, mask, degb2,
      W_in.astype(bf16), b_in[None, :], W1.astype(bf16), b1[None, :], W2.astype(bf16),
      W_ih.astype(bf16), W_hh.astype(bf16), b_ih[None, :], b_hh[None, :],
      Wo1.astype(bf16), bo1[None, :], wo2p, bo2p)

    return out[:, :Wo2.shape[1]].reshape(_B, _N, Wo2.shape[1])
